# baseline (speedup 1.0000x reference)
import dataclasses
import functools

import jax
import jax.numpy as jnp
from jax import lax
from jax.experimental import pallas as pl
from jax.experimental.pallas import tpu as pltpu
from jax.experimental.pallas import tpu_sc as plsc

N_REAL = 10000
N_PAD = 10240
D = 128
NC, NS = 2, 16
NW = NC * NS
EB = 128
E_REAL = 320000
NH = 2
NBH = 40
NB = NH * NBH
N_CHUNKS = NW * NB
E_PAD = N_CHUNKS * EB
EPW = E_PAD // NW
ROWS_PER_TILE = N_PAD // NS

_mesh = plsc.VectorSubcoreMesh(core_axis_name="c", subcore_axis_name="s")

_sc_params = pltpu.CompilerParams()
if "needs_layout_passes" in pltpu.CompilerParams.__dataclass_fields__:
    _sc_params = dataclasses.replace(_sc_params, needs_layout_passes=False)


@functools.partial(
    pl.kernel,
    mesh=_mesh,
    out_type=jax.ShapeDtypeStruct((NW, N_PAD), jnp.float32),
    compiler_params=_sc_params,
    scratch_types=[
        pltpu.VMEM((EPW,), jnp.int32),
        pltpu.VMEM((N_PAD,), jnp.float32),
        pltpu.SemaphoreType.DMA,
    ],
)
def _deg_kernel(dst_hbm, out_hbm, idx_v, hist_v, sem):
    c = lax.axis_index("c")
    s = lax.axis_index("s")
    w = c * NS + s
    cp = pltpu.async_copy(dst_hbm.at[w], idx_v, sem)
    z16 = jnp.zeros((16,), jnp.float32)

    @pl.loop(0, N_PAD // 16)
    def _(i):
        hist_v[pl.ds(i * 16, 16)] = z16

    cp.wait()
    ones16 = jnp.ones((16,), jnp.float32)

    @pl.loop(0, EPW // 16)
    def _(t):
        idx = idx_v[pl.ds(t * 16, 16)]
        plsc.addupdate_scatter(hist_v, [idx], ones16)

    pltpu.sync_copy(hist_v, out_hbm.at[w])


@functools.partial(
    pl.kernel,
    mesh=_mesh,
    out_type=jax.ShapeDtypeStruct((NC, N_PAD, D), jnp.float32),
    scratch_types=[
        pltpu.VMEM((NBH, EB), jnp.int32),
        pltpu.VMEM((NBH, EB), jnp.int32),
        pltpu.VMEM((EB, D), jnp.float32),
        pltpu.VMEM((EB, D), jnp.float32),
        pltpu.VMEM_SHARED((N_PAD, D), jnp.float32),
        pltpu.SemaphoreType.DMA,
        pltpu.SemaphoreType.DMA,
        pltpu.SemaphoreType.DMA,
    ],
)
def _scatter_kernel(g_hbm, src_hbm, dst_hbm, zeros_hbm, out_hbm,
                    sidx, didx, rows_a, rows_b, acc, sem, sem_a, sem_b):
    c = lax.axis_index("c")
    s = lax.axis_index("s")
    w = c * NS + s
    cp_s = pltpu.async_copy(src_hbm.at[w, 0], sidx, sem)
    cp_d = pltpu.async_copy(dst_hbm.at[w, 0], didx, sem)
    pltpu.sync_copy(zeros_hbm, acc.at[pl.ds(s * ROWS_PER_TILE, ROWS_PER_TILE)])
    plsc.subcore_barrier()

    def gather(j, buf):
        pltpu.sync_copy(g_hbm.at[sidx.at[j]], buf)

    def scat(j, buf, sem_g):
        pltpu.async_copy(buf, acc.at[didx.at[j]], sem_g, add=True)

    def wait_scat(j, buf, sem_g):
        pltpu.make_async_copy(buf, acc.at[didx.at[j]], sem_g).wait()

    for h in range(NH):
        cp_s.wait()
        cp_d.wait()
        gather(0, rows_a)
        scat(0, rows_a, sem_a)

        @pl.loop(0, NBH // 2 - 1)
        def _(p):
            gather(2 * p + 1, rows_b)
            wait_scat(2 * p, rows_a, sem_a)
            scat(2 * p + 1, rows_b, sem_b)
            gather(2 * p + 2, rows_a)
            wait_scat(2 * p + 1, rows_b, sem_b)
            scat(2 * p + 2, rows_a, sem_a)

        gather(NBH - 1, rows_b)
        wait_scat(NBH - 2, rows_a, sem_a)
        scat(NBH - 1, rows_b, sem_b)
        wait_scat(NBH - 1, rows_b, sem_b)
        if h + 1 < NH:
            cp_s = pltpu.async_copy(src_hbm.at[w, h + 1], sidx, sem)
            cp_d = pltpu.async_copy(dst_hbm.at[w, h + 1], didx, sem)

    plsc.subcore_barrier()
    pltpu.sync_copy(acc.at[pl.ds(s * ROWS_PER_TILE, ROWS_PER_TILE)],
                    out_hbm.at[c, pl.ds(s * ROWS_PER_TILE, ROWS_PER_TILE)])


_R = 1000
_NBLK = N_REAL // _R


def _first_body(x_ref, w_ref, parts_ref, h_ref, g_ref, dinv_ref):
    h = jnp.dot(x_ref[...], w_ref[...], preferred_element_type=jnp.float32)
    deg = jnp.sum(parts_ref[...], axis=1) + 1.0
    dinv = lax.rsqrt(deg)[:, None]
    h_ref[...] = h
    g_ref[...] = h * dinv
    dinv_ref[...] = dinv


def _first(x, w, parts):
    return pl.pallas_call(
        _first_body,
        grid=(_NBLK,),
        in_specs=[pl.BlockSpec((_R, D), lambda i: (i, 0)),
                  pl.BlockSpec((D, D), lambda i: (0, 0)),
                  pl.BlockSpec((_R, NW), lambda i: (i, 0))],
        out_specs=[pl.BlockSpec((_R, D), lambda i: (i, 0)),
                   pl.BlockSpec((_R, D), lambda i: (i, 0)),
                   pl.BlockSpec((_R, 1), lambda i: (i, 0))],
        out_shape=[jax.ShapeDtypeStruct((N_REAL, D), jnp.float32),
                   jax.ShapeDtypeStruct((N_REAL, D), jnp.float32),
                   jax.ShapeDtypeStruct((N_REAL, 1), jnp.float32)],
    )(x, w, parts)


def _mid_body(h1_ref, acc_ref, dinv_ref, b_ref, w_ref, h2_ref, g2_ref):
    dinv = dinv_ref[...]
    accs = acc_ref[...]
    h1 = jnp.maximum(dinv * (accs[0] + accs[1]) + (dinv * dinv) * h1_ref[...]
                     + b_ref[...], 0.0)
    h2 = jnp.dot(h1, w_ref[...], preferred_element_type=jnp.float32)
    h2_ref[...] = h2
    g2_ref[...] = h2 * dinv


def _mid(h1raw, acc, dinv, b1, w2):
    return pl.pallas_call(
        _mid_body,
        grid=(_NBLK,),
        in_specs=[pl.BlockSpec((_R, D), lambda i: (i, 0)),
                  pl.BlockSpec((NC, _R, D), lambda i: (0, i, 0)),
                  pl.BlockSpec((_R, 1), lambda i: (i, 0)),
                  pl.BlockSpec((1, D), lambda i: (0, 0)),
                  pl.BlockSpec((D, D), lambda i: (0, 0))],
        out_specs=[pl.BlockSpec((_R, D), lambda i: (i, 0)),
                   pl.BlockSpec((_R, D), lambda i: (i, 0))],
        out_shape=[jax.ShapeDtypeStruct((N_REAL, D), jnp.float32),
                   jax.ShapeDtypeStruct((N_REAL, D), jnp.float32)],
    )(h1raw, acc, dinv, b1, w2)


def _final_body(h2_ref, acc_ref, dinv_ref, b_ref, o_ref):
    dinv = dinv_ref[...]
    accs = acc_ref[...]
    o_ref[...] = jnp.maximum(dinv * (accs[0] + accs[1])
                             + (dinv * dinv) * h2_ref[...] + b_ref[...], 0.0)


def _final(h2raw, acc, dinv, b2):
    return pl.pallas_call(
        _final_body,
        grid=(_NBLK,),
        in_specs=[pl.BlockSpec((_R, D), lambda i: (i, 0)),
                  pl.BlockSpec((NC, _R, D), lambda i: (0, i, 0)),
                  pl.BlockSpec((_R, 1), lambda i: (i, 0)),
                  pl.BlockSpec((1, D), lambda i: (0, 0))],
        out_specs=pl.BlockSpec((_R, D), lambda i: (i, 0)),
        out_shape=jax.ShapeDtypeStruct((N_REAL, D), jnp.float32),
    )(h2raw, acc, dinv, b2)


def kernel(x, edge_index, W1, b1, W2, b2):
    ei = edge_index.astype(jnp.int32)
    pad = E_PAD - E_REAL
    src_p = jnp.concatenate(
        [ei[0], jnp.arange(pad, dtype=jnp.int32) % N_REAL])
    dst_p = jnp.concatenate(
        [ei[1], N_REAL + (jnp.arange(pad, dtype=jnp.int32) % (N_PAD - N_REAL))])
    src_w = src_p.reshape(NW, NH, NBH, EB)
    dst_w = dst_p.reshape(NW, NH, NBH, EB)
    dst_flat = dst_p.reshape(NW, EPW)
    zeros_tile = jnp.zeros((ROWS_PER_TILE, D), jnp.float32)

    b1r = b1.reshape(1, D)
    b2r = b2.reshape(1, D)

    deg_parts = _deg_kernel(dst_flat)
    h1raw, g1, dinv = _first(x, W1, deg_parts.T)
    acc1 = _scatter_kernel(g1, src_w, dst_w, zeros_tile)
    h2raw, g2 = _mid(h1raw, acc1, dinv, b1r, W2)
    acc2 = _scatter_kernel(g2, src_w, dst_w, zeros_tile)
    return _final(h2raw, acc2, dinv, b2r)


# speedup vs baseline: 1.0147x; 1.0147x over previous
import dataclasses
import functools

import jax
import jax.numpy as jnp
from jax import lax
from jax.experimental import pallas as pl
from jax.experimental.pallas import tpu as pltpu
from jax.experimental.pallas import tpu_sc as plsc

N_REAL = 10000
N_PAD = 10240
D = 128
NC, NS = 2, 16
NW = NC * NS
EB = 128
E_REAL = 320000
NH = 2
NBH = 40
NB = NH * NBH
N_CHUNKS = NW * NB
E_PAD = N_CHUNKS * EB
EPW = E_PAD // NW
ROWS_PER_TILE = N_PAD // NS

_mesh = plsc.VectorSubcoreMesh(core_axis_name="c", subcore_axis_name="s")

_sc_params = pltpu.CompilerParams()
if "needs_layout_passes" in pltpu.CompilerParams.__dataclass_fields__:
    _sc_params = dataclasses.replace(_sc_params, needs_layout_passes=False)


@functools.partial(
    pl.kernel,
    mesh=_mesh,
    out_type=jax.ShapeDtypeStruct((NW, N_PAD), jnp.float32),
    compiler_params=_sc_params,
    scratch_types=[
        pltpu.VMEM((EPW,), jnp.int32),
        pltpu.VMEM((N_PAD,), jnp.float32),
        pltpu.SemaphoreType.DMA,
    ],
)
def _deg_kernel(dst_hbm, out_hbm, idx_v, hist_v, sem):
    c = lax.axis_index("c")
    s = lax.axis_index("s")
    w = c * NS + s
    cp = pltpu.async_copy(dst_hbm.at[w], idx_v, sem)
    z16 = jnp.zeros((16,), jnp.float32)

    @pl.loop(0, N_PAD // 16)
    def _(i):
        hist_v[pl.ds(i * 16, 16)] = z16

    cp.wait()
    ones16 = jnp.ones((16,), jnp.float32)

    @pl.loop(0, EPW // 16)
    def _(t):
        idx = idx_v[pl.ds(t * 16, 16)]
        plsc.addupdate_scatter(hist_v, [idx], ones16)

    pltpu.sync_copy(hist_v, out_hbm.at[w])


@functools.partial(
    pl.kernel,
    mesh=_mesh,
    out_type=jax.ShapeDtypeStruct((NC, N_PAD, D), jnp.float32),
    scratch_types=[
        pltpu.VMEM((NBH, EB), jnp.int32),
        pltpu.VMEM((NBH, EB), jnp.int32),
        pltpu.VMEM((EB, D), jnp.float32),
        pltpu.VMEM((EB, D), jnp.float32),
        pltpu.VMEM_SHARED((N_PAD, D), jnp.float32),
        pltpu.SemaphoreType.DMA,
        pltpu.SemaphoreType.DMA,
        pltpu.SemaphoreType.DMA,
    ],
)
def _scatter_kernel(g_hbm, src_hbm, dst_hbm, zeros_hbm, out_hbm,
                    sidx, didx, rows_a, rows_b, acc, sem, sem_a, sem_b):
    c = lax.axis_index("c")
    s = lax.axis_index("s")
    w = c * NS + s
    cp_s = pltpu.async_copy(src_hbm.at[w, 0], sidx, sem)
    cp_d = pltpu.async_copy(dst_hbm.at[w, 0], didx, sem)
    pltpu.sync_copy(zeros_hbm, acc.at[pl.ds(s * ROWS_PER_TILE, ROWS_PER_TILE)])
    plsc.subcore_barrier()

    def gather(j, buf):
        pltpu.sync_copy(g_hbm.at[sidx.at[j]], buf)

    def scat(j, buf, sem_g):
        pltpu.async_copy(buf, acc.at[didx.at[j]], sem_g, add=True)

    def wait_scat(j, buf, sem_g):
        pltpu.make_async_copy(buf, acc.at[didx.at[j]], sem_g).wait()

    for h in range(NH):
        cp_s.wait()
        cp_d.wait()
        gather(0, rows_a)
        scat(0, rows_a, sem_a)

        @pl.loop(0, NBH // 2 - 1)
        def _(p):
            gather(2 * p + 1, rows_b)
            wait_scat(2 * p, rows_a, sem_a)
            scat(2 * p + 1, rows_b, sem_b)
            gather(2 * p + 2, rows_a)
            wait_scat(2 * p + 1, rows_b, sem_b)
            scat(2 * p + 2, rows_a, sem_a)

        gather(NBH - 1, rows_b)
        wait_scat(NBH - 2, rows_a, sem_a)
        scat(NBH - 1, rows_b, sem_b)
        wait_scat(NBH - 1, rows_b, sem_b)
        if h + 1 < NH:
            cp_s = pltpu.async_copy(src_hbm.at[w, h + 1], sidx, sem)
            cp_d = pltpu.async_copy(dst_hbm.at[w, h + 1], didx, sem)

    plsc.subcore_barrier()
    pltpu.sync_copy(acc.at[pl.ds(s * ROWS_PER_TILE, ROWS_PER_TILE)],
                    out_hbm.at[c, pl.ds(s * ROWS_PER_TILE, ROWS_PER_TILE)])


_R = 2000
_NBLK = N_REAL // _R


def _first_body(x_ref, w_ref, parts_ref, h_ref, g_ref, dinv_ref):
    h = jnp.dot(x_ref[...], w_ref[...], preferred_element_type=jnp.float32)
    deg = jnp.sum(parts_ref[...], axis=1) + 1.0
    dinv = lax.rsqrt(deg)[:, None]
    h_ref[...] = h
    g_ref[...] = h * dinv
    dinv_ref[...] = dinv


def _first(x, w, parts):
    return pl.pallas_call(
        _first_body,
        grid=(_NBLK,),
        in_specs=[pl.BlockSpec((_R, D), lambda i: (i, 0)),
                  pl.BlockSpec((D, D), lambda i: (0, 0)),
                  pl.BlockSpec((_R, NW), lambda i: (i, 0))],
        out_specs=[pl.BlockSpec((_R, D), lambda i: (i, 0)),
                   pl.BlockSpec((_R, D), lambda i: (i, 0)),
                   pl.BlockSpec((_R, 1), lambda i: (i, 0))],
        out_shape=[jax.ShapeDtypeStruct((N_REAL, D), jnp.float32),
                   jax.ShapeDtypeStruct((N_REAL, D), jnp.float32),
                   jax.ShapeDtypeStruct((N_REAL, 1), jnp.float32)],
    )(x, w, parts)


def _mid_body(h1_ref, acc_ref, dinv_ref, b_ref, w_ref, h2_ref, g2_ref):
    dinv = dinv_ref[...]
    accs = acc_ref[...]
    h1 = jnp.maximum(dinv * (accs[0] + accs[1]) + (dinv * dinv) * h1_ref[...]
                     + b_ref[...], 0.0)
    h2 = jnp.dot(h1, w_ref[...], preferred_element_type=jnp.float32)
    h2_ref[...] = h2
    g2_ref[...] = h2 * dinv


def _mid(h1raw, acc, dinv, b1, w2):
    return pl.pallas_call(
        _mid_body,
        grid=(_NBLK,),
        in_specs=[pl.BlockSpec((_R, D), lambda i: (i, 0)),
                  pl.BlockSpec((NC, _R, D), lambda i: (0, i, 0)),
                  pl.BlockSpec((_R, 1), lambda i: (i, 0)),
                  pl.BlockSpec((1, D), lambda i: (0, 0)),
                  pl.BlockSpec((D, D), lambda i: (0, 0))],
        out_specs=[pl.BlockSpec((_R, D), lambda i: (i, 0)),
                   pl.BlockSpec((_R, D), lambda i: (i, 0))],
        out_shape=[jax.ShapeDtypeStruct((N_REAL, D), jnp.float32),
                   jax.ShapeDtypeStruct((N_REAL, D), jnp.float32)],
    )(h1raw, acc, dinv, b1, w2)


def _final_body(h2_ref, acc_ref, dinv_ref, b_ref, o_ref):
    dinv = dinv_ref[...]
    accs = acc_ref[...]
    o_ref[...] = jnp.maximum(dinv * (accs[0] + accs[1])
                             + (dinv * dinv) * h2_ref[...] + b_ref[...], 0.0)


def _final(h2raw, acc, dinv, b2):
    return pl.pallas_call(
        _final_body,
        grid=(_NBLK,),
        in_specs=[pl.BlockSpec((_R, D), lambda i: (i, 0)),
                  pl.BlockSpec((NC, _R, D), lambda i: (0, i, 0)),
                  pl.BlockSpec((_R, 1), lambda i: (i, 0)),
                  pl.BlockSpec((1, D), lambda i: (0, 0))],
        out_specs=pl.BlockSpec((_R, D), lambda i: (i, 0)),
        out_shape=jax.ShapeDtypeStruct((N_REAL, D), jnp.float32),
    )(h2raw, acc, dinv, b2)


def kernel(x, edge_index, W1, b1, W2, b2):
    ei = edge_index.astype(jnp.int32)
    pad = E_PAD - E_REAL
    pad_iota = (lax.broadcasted_iota(jnp.int32, (pad // EB, EB), 0) * EB
                + lax.broadcasted_iota(jnp.int32, (pad // EB, EB), 1))
    src_p = jnp.concatenate(
        [ei[0].reshape(E_REAL // EB, EB), pad_iota % N_REAL])
    dst_p = jnp.concatenate(
        [ei[1].reshape(E_REAL // EB, EB), N_REAL + pad_iota % (N_PAD - N_REAL)])
    src_w = src_p.reshape(NW, NH, NBH, EB)
    dst_w = dst_p.reshape(NW, NH, NBH, EB)
    dst_flat = dst_p.reshape(NW, EPW)
    zeros_tile = jnp.zeros((ROWS_PER_TILE, D), jnp.float32)

    b1r = b1.reshape(1, D)
    b2r = b2.reshape(1, D)

    deg_parts = _deg_kernel(dst_flat)
    h1raw, g1, dinv = _first(x, W1, deg_parts.T)
    acc1 = _scatter_kernel(g1, src_w, dst_w, zeros_tile)
    h2raw, g2 = _mid(h1raw, acc1, dinv, b1r, W2)
    acc2 = _scatter_kernel(g2, src_w, dst_w, zeros_tile)
    return _final(h2raw, acc2, dinv, b2r)


# speedup vs baseline: 1.0352x; 1.0202x over previous
import dataclasses
import functools

import jax
import jax.numpy as jnp
from jax import lax
from jax.experimental import pallas as pl
from jax.experimental.pallas import tpu as pltpu
from jax.experimental.pallas import tpu_sc as plsc

N_REAL = 10000
N_PAD = 10240
D = 128
NC, NS = 2, 16
NW = NC * NS
EB = 128
E_REAL = 320000
NB_MAIN = 72
N_MAIN = NW * NB_MAIN * EB
STAGES = (40, 32)
NBX = 8
EPW = (NB_MAIN + NBX) * EB
ROWS_PER_TILE = N_PAD // NS

_mesh = plsc.VectorSubcoreMesh(core_axis_name="c", subcore_axis_name="s")

_sc_params = pltpu.CompilerParams()
if "needs_layout_passes" in pltpu.CompilerParams.__dataclass_fields__:
    _sc_params = dataclasses.replace(_sc_params, needs_layout_passes=False)


@functools.partial(
    pl.kernel,
    mesh=_mesh,
    out_type=jax.ShapeDtypeStruct((NW, N_PAD), jnp.float32),
    compiler_params=_sc_params,
    scratch_types=[
        pltpu.VMEM((EPW,), jnp.int32),
        pltpu.VMEM((N_PAD,), jnp.float32),
        pltpu.SemaphoreType.DMA,
    ],
)
def _deg_kernel(dst_hbm, extra_hbm, out_hbm, idx_v, hist_v, sem):
    c = lax.axis_index("c")
    s = lax.axis_index("s")
    w = c * NS + s
    cp = pltpu.async_copy(dst_hbm.at[pl.ds(w * (NB_MAIN * EB), NB_MAIN * EB)],
                          idx_v.at[pl.ds(0, NB_MAIN * EB)], sem)
    cp2 = pltpu.async_copy(extra_hbm.at[w],
                           idx_v.at[pl.ds(NB_MAIN * EB, NBX * EB)], sem)
    z16 = jnp.zeros((16,), jnp.float32)

    @pl.loop(0, N_PAD // 16)
    def _(i):
        hist_v[pl.ds(i * 16, 16)] = z16

    cp.wait()
    cp2.wait()
    ones16 = jnp.ones((16,), jnp.float32)

    @pl.loop(0, EPW // 16)
    def _(t):
        idx = idx_v[pl.ds(t * 16, 16)]
        plsc.addupdate_scatter(hist_v, [idx], ones16)

    pltpu.sync_copy(hist_v, out_hbm.at[w])


@functools.partial(
    pl.kernel,
    mesh=_mesh,
    out_type=jax.ShapeDtypeStruct((NC, N_PAD, D), jnp.float32),
    scratch_types=[
        pltpu.VMEM((STAGES[0], EB), jnp.int32),
        pltpu.VMEM((STAGES[0], EB), jnp.int32),
        pltpu.VMEM((EB, D), jnp.float32),
        pltpu.VMEM((EB, D), jnp.float32),
        pltpu.VMEM_SHARED((N_PAD, D), jnp.float32),
        pltpu.SemaphoreType.DMA,
        pltpu.SemaphoreType.DMA,
        pltpu.SemaphoreType.DMA,
    ],
)
def _scatter_kernel(g_hbm, src_hbm, dst_hbm, xsrc_hbm, xdst_hbm, zeros_hbm,
                    out_hbm, sidx, didx, rows_a, rows_b, acc,
                    sem, sem_a, sem_b):
    c = lax.axis_index("c")
    s = lax.axis_index("s")
    w = c * NS + s
    base = w * NB_MAIN

    def load_idx(off, n):
        return (pltpu.async_copy(src_hbm.at[pl.ds(base + off, n)],
                                 sidx.at[pl.ds(0, n)], sem),
                pltpu.async_copy(dst_hbm.at[pl.ds(base + off, n)],
                                 didx.at[pl.ds(0, n)], sem))

    cp_s, cp_d = load_idx(0, STAGES[0])
    pltpu.sync_copy(zeros_hbm, acc.at[pl.ds(s * ROWS_PER_TILE, ROWS_PER_TILE)])
    plsc.subcore_barrier()

    def gather(j, buf):
        pltpu.sync_copy(g_hbm.at[sidx.at[j]], buf)

    def scat(j, buf, sem_g):
        pltpu.async_copy(buf, acc.at[didx.at[j]], sem_g, add=True)

    def wait_scat(j, buf, sem_g):
        pltpu.make_async_copy(buf, acc.at[didx.at[j]], sem_g).wait()

    for h, nbh in enumerate(STAGES + (NBX,)):
        cp_s.wait()
        cp_d.wait()
        gather(0, rows_a)
        scat(0, rows_a, sem_a)

        @pl.loop(0, nbh // 2 - 1)
        def _(p):
            gather(2 * p + 1, rows_b)
            wait_scat(2 * p, rows_a, sem_a)
            scat(2 * p + 1, rows_b, sem_b)
            gather(2 * p + 2, rows_a)
            wait_scat(2 * p + 1, rows_b, sem_b)
            scat(2 * p + 2, rows_a, sem_a)

        gather(nbh - 1, rows_b)
        wait_scat(nbh - 2, rows_a, sem_a)
        scat(nbh - 1, rows_b, sem_b)
        wait_scat(nbh - 1, rows_b, sem_b)
        if h == 0:
            cp_s, cp_d = load_idx(STAGES[0], STAGES[1])
        elif h == 1:
            cp_s = pltpu.async_copy(xsrc_hbm.at[w],
                                    sidx.at[pl.ds(0, NBX)], sem)
            cp_d = pltpu.async_copy(xdst_hbm.at[w],
                                    didx.at[pl.ds(0, NBX)], sem)

    plsc.subcore_barrier()
    pltpu.sync_copy(acc.at[pl.ds(s * ROWS_PER_TILE, ROWS_PER_TILE)],
                    out_hbm.at[c, pl.ds(s * ROWS_PER_TILE, ROWS_PER_TILE)])


_R = 2000
_NBLK = N_REAL // _R


def _first_body(x_ref, w_ref, parts_ref, h_ref, g_ref, dinv_ref):
    h = jnp.dot(x_ref[...], w_ref[...], preferred_element_type=jnp.float32)
    deg = jnp.sum(parts_ref[...], axis=1) + 1.0
    dinv = lax.rsqrt(deg)[:, None]
    h_ref[...] = h
    g_ref[...] = h * dinv
    dinv_ref[...] = dinv


def _first(x, w, parts):
    return pl.pallas_call(
        _first_body,
        grid=(_NBLK,),
        in_specs=[pl.BlockSpec((_R, D), lambda i: (i, 0)),
                  pl.BlockSpec((D, D), lambda i: (0, 0)),
                  pl.BlockSpec((_R, NW), lambda i: (i, 0))],
        out_specs=[pl.BlockSpec((_R, D), lambda i: (i, 0)),
                   pl.BlockSpec((_R, D), lambda i: (i, 0)),
                   pl.BlockSpec((_R, 1), lambda i: (i, 0))],
        out_shape=[jax.ShapeDtypeStruct((N_REAL, D), jnp.float32),
                   jax.ShapeDtypeStruct((N_REAL, D), jnp.float32),
                   jax.ShapeDtypeStruct((N_REAL, 1), jnp.float32)],
    )(x, w, parts)


def _mid_body(h1_ref, acc_ref, dinv_ref, b_ref, w_ref, h2_ref, g2_ref):
    dinv = dinv_ref[...]
    accs = acc_ref[...]
    h1 = jnp.maximum(dinv * (accs[0] + accs[1]) + (dinv * dinv) * h1_ref[...]
                     + b_ref[...], 0.0)
    h2 = jnp.dot(h1, w_ref[...], preferred_element_type=jnp.float32)
    h2_ref[...] = h2
    g2_ref[...] = h2 * dinv


def _mid(h1raw, acc, dinv, b1, w2):
    return pl.pallas_call(
        _mid_body,
        grid=(_NBLK,),
        in_specs=[pl.BlockSpec((_R, D), lambda i: (i, 0)),
                  pl.BlockSpec((NC, _R, D), lambda i: (0, i, 0)),
                  pl.BlockSpec((_R, 1), lambda i: (i, 0)),
                  pl.BlockSpec((1, D), lambda i: (0, 0)),
                  pl.BlockSpec((D, D), lambda i: (0, 0))],
        out_specs=[pl.BlockSpec((_R, D), lambda i: (i, 0)),
                   pl.BlockSpec((_R, D), lambda i: (i, 0))],
        out_shape=[jax.ShapeDtypeStruct((N_REAL, D), jnp.float32),
                   jax.ShapeDtypeStruct((N_REAL, D), jnp.float32)],
    )(h1raw, acc, dinv, b1, w2)


def _final_body(h2_ref, acc_ref, dinv_ref, b_ref, o_ref):
    dinv = dinv_ref[...]
    accs = acc_ref[...]
    o_ref[...] = jnp.maximum(dinv * (accs[0] + accs[1])
                             + (dinv * dinv) * h2_ref[...] + b_ref[...], 0.0)


def _final(h2raw, acc, dinv, b2):
    return pl.pallas_call(
        _final_body,
        grid=(_NBLK,),
        in_specs=[pl.BlockSpec((_R, D), lambda i: (i, 0)),
                  pl.BlockSpec((NC, _R, D), lambda i: (0, i, 0)),
                  pl.BlockSpec((_R, 1), lambda i: (i, 0)),
                  pl.BlockSpec((1, D), lambda i: (0, 0))],
        out_specs=pl.BlockSpec((_R, D), lambda i: (i, 0)),
        out_shape=jax.ShapeDtypeStruct((N_REAL, D), jnp.float32),
    )(h2raw, acc, dinv, b2)


def kernel(x, edge_index, W1, b1, W2, b2):
    ei = edge_index.astype(jnp.int32)
    src_main = ei[0, :N_MAIN].reshape(NW * NB_MAIN, EB)
    dst_main = ei[1, :N_MAIN].reshape(NW * NB_MAIN, EB)
    dst_main_flat = ei[1, :N_MAIN]
    npad = NW * NBX * EB - (E_REAL - N_MAIN)
    pad_iota = (lax.broadcasted_iota(jnp.int32, (npad // EB, EB), 0) * EB
                + lax.broadcasted_iota(jnp.int32, (npad // EB, EB), 1))
    tail = (E_REAL - N_MAIN) // EB
    xsrc = jnp.concatenate(
        [ei[0, N_MAIN:].reshape(tail, EB),
         pad_iota % N_REAL]).reshape(NW, NBX, EB)
    xdst = jnp.concatenate(
        [ei[1, N_MAIN:].reshape(tail, EB),
         N_REAL + pad_iota % (N_PAD - N_REAL)]).reshape(NW, NBX, EB)
    xdst_flat = xdst.reshape(NW, NBX * EB)
    zeros_tile = jnp.zeros((ROWS_PER_TILE, D), jnp.float32)

    b1r = b1.reshape(1, D)
    b2r = b2.reshape(1, D)

    deg_parts = _deg_kernel(dst_main_flat, xdst_flat)
    h1raw, g1, dinv = _first(x, W1, deg_parts.T)
    acc1 = _scatter_kernel(g1, src_main, dst_main, xsrc, xdst, zeros_tile)
    h2raw, g2 = _mid(h1raw, acc1, dinv, b1r, W2)
    acc2 = _scatter_kernel(g2, src_main, dst_main, xsrc, xdst, zeros_tile)
    return _final(h2raw, acc2, dinv, b2r)
